# baseline (speedup 1.0000x reference)
import functools

import jax
import jax.numpy as jnp
from jax import lax
from jax.experimental import pallas as pl
from jax.experimental.pallas import tpu as pltpu
from jax.experimental.pallas import tpu_sc as plsc

_S = 64
_RAD = 0.05
_K = 8
_P = 2000
_ROWS = 4
_PIX = _ROWS * _S
_GRP = _P // 16
_INV32 = 0.03125
_EPS = 1e-3


def _sc_rasterize(verts):
    mesh = plsc.VectorSubcoreMesh(
        core_axis_name="c", subcore_axis_name="s", num_cores=2,
        num_subcores=16)

    @functools.partial(
        pl.kernel,
        mesh=mesh,
        compiler_params=pltpu.CompilerParams(needs_layout_passes=False),
        out_type=(
            jax.ShapeDtypeStruct((2, _S, _S, _K), jnp.int32),
            jax.ShapeDtypeStruct((2, _S, _S, _K), jnp.float32),
            jax.ShapeDtypeStruct((2, _S, _S, _K), jnp.float32),
        ),
        scratch_types=[
            pltpu.VMEM((3 * _P,), jnp.float32),
            pltpu.SemaphoreType.DMA,
            pltpu.SemaphoreType.DMA,
            pltpu.SemaphoreType.DMA,
            pltpu.SemaphoreType.DMA,
            pltpu.VMEM((16,), jnp.float32),
            pltpu.VMEM((16,), jnp.float32),
            pltpu.VMEM((16,), jnp.float32),
            pltpu.VMEM((16,), jnp.int32),
            pltpu.VMEM((_PIX * _K,), jnp.float32),
            pltpu.VMEM((_PIX * _K,), jnp.int32),
            pltpu.VMEM((_PIX * _K,), jnp.float32),
            pltpu.VMEM((_ROWS, _S, _K), jnp.float32),
            pltpu.VMEM((_ROWS, _S, _K), jnp.int32),
            pltpu.VMEM((_ROWS, _S, _K), jnp.float32),
        ],
    )
    def k(verts_hbm, oi_hbm, oz_hbm, od_hbm,
          pv, dsem, osem1, osem2, osem3, cx, cy, cz, ci, z8, i8, d8,
          zs3, is3, ds3):
        c = lax.axis_index("c")
        s = lax.axis_index("s")
        b = s & 1
        band = (s >> 1) + c * 8
        row0 = band * _ROWS

        pbase = b * (3 * _P)
        in_dma = pltpu.async_copy(verts_hbm.at[pl.ds(pbase, 3 * _P)], pv, dsem)

        lane = lax.iota(jnp.int32, 16)
        lane3 = lane * 3
        lm4 = lane & 3
        ld4 = lane >> 2
        rows_f = lax.convert_element_type(row0 + ld4, jnp.float32)
        pyv = 1.0 - (rows_f + 0.5) * _INV32
        row0_f = rows_f - lax.convert_element_type(ld4, jnp.float32)
        ymin = 1.0 - (row0_f + 3.5) * _INV32 - (_RAD + _EPS)
        ymax = 1.0 - (row0_f + 0.5) * _INV32 + (_RAD + _EPS)
        xlo = jnp.full((16,), -0.984375 - (_RAD + _EPS), jnp.float32)
        xhi = jnp.full((16,), 0.984375 + (_RAD + _EPS), jnp.float32)
        inf_v = jnp.full((16,), jnp.inf, jnp.float32)

        def init_body(i, carry):
            z8[pl.ds(i * 16, 16)] = inf_v
            return carry
        lax.fori_loop(0, _PIX * _K // 16, init_body, 0)
        in_dma.wait()

        r2v = jnp.full((16,), _RAD * _RAD, jnp.float32)

        def cand_body(i, carry):
            ib = lax.broadcast_in_dim(i, (16,), ())
            xs = plsc.load_gather(cx.at[...], [ib])
            ys = plsc.load_gather(cy.at[...], [ib])
            zs = plsc.load_gather(cz.at[...], [ib])
            iv = plsc.load_gather(ci.at[...], [ib])
            u = (1.0 - xs - _RAD) * 32.0 - 0.5 - _EPS
            ti = lax.convert_element_type(u, jnp.int32)
            ti = ti - jnp.where(lax.convert_element_type(ti, jnp.float32) > u,
                                jnp.int32(1), jnp.int32(0))
            cols = ti + 1 + lm4
            colf = lax.convert_element_type(cols, jnp.float32)
            pxv = 1.0 - (colf + 0.5) * _INV32
            dx = pxv - xs
            dy = pyv - ys
            d2 = dx * dx + dy * dy
            valid = (d2 < r2v) & (cols >= 0) & (cols < _S)
            nv = plsc.all_reduce_population_count(valid)[0]

            @pl.when(nv > 0)
            def _insert():
                addr = jnp.where(valid, (ld4 * _S + cols) * _K, jnp.int32(0))
                candz = zs
                candi = iv
                candd = d2
                for j in range(_K):
                    a = addr + j
                    curz = plsc.load_gather(z8.at[...], [a])
                    curi = plsc.load_gather(i8.at[...], [a])
                    curd = plsc.load_gather(d8.at[...], [a])
                    swap = candz < curz
                    newz = jnp.where(swap, candz, curz)
                    candz = jnp.where(swap, curz, candz)
                    newi = jnp.where(swap, candi, curi)
                    candi = jnp.where(swap, curi, candi)
                    newd = jnp.where(swap, candd, curd)
                    candd = jnp.where(swap, curd, candd)
                    plsc.store_scatter(z8.at[...], [a], newz, mask=valid)
                    plsc.store_scatter(i8.at[...], [a], newi, mask=valid)
                    plsc.store_scatter(d8.at[...], [a], newd, mask=valid)
                candz, candi, candd = None, None, None
            return carry

        def g_body(g, carry):
            off = g * 16
            i3 = off * 3 + lane3
            vx = plsc.load_gather(pv.at[...], [i3])
            vy = plsc.load_gather(pv.at[...], [i3 + 1])
            vz = plsc.load_gather(pv.at[...], [i3 + 2])
            zsafe = jnp.where(jnp.abs(vz) < 1e-4, jnp.float32(1e-4), vz)
            m = ((vz > 0.0)
                 & (vy > ymin * zsafe) & (vy < ymax * zsafe)
                 & (vx > xlo * zsafe) & (vx < xhi * zsafe))
            n = plsc.all_reduce_population_count(m)[0]

            @pl.when(n > 0)
            def _emit():
                xn = vx / zsafe
                yn = vy / zsafe
                plsc.store_compressed(cx.at[...], xn, mask=m)
                plsc.store_compressed(cy.at[...], yn, mask=m)
                plsc.store_compressed(cz.at[...], vz, mask=m)
                plsc.store_compressed(ci.at[...], off + lane, mask=m)
                lax.fori_loop(0, n, cand_body, 0)
            return carry

        lax.fori_loop(0, _GRP, g_body, 0)

        kk_s = lane & (_K - 1)

        def fin_body(i, carry):
            sl = pl.ds(i * 16, 16)
            zv = z8[sl]
            empty = zv >= 3e38
            flat = i * 16 + lane
            pix = flat >> 3
            idxs = [pix >> 6, pix & (_S - 1), kk_s]
            plsc.store_scatter(zs3.at[...], idxs,
                               jnp.where(empty, jnp.float32(-1.0), zv))
            plsc.store_scatter(is3.at[...], idxs,
                               jnp.where(empty, jnp.int32(-1), i8[sl]))
            plsc.store_scatter(ds3.at[...], idxs,
                               jnp.where(empty, jnp.float32(-1.0), d8[sl]))
            return carry
        lax.fori_loop(0, _PIX * _K // 16, fin_body, 0)

        o1 = pltpu.async_copy(is3, oi_hbm.at[b, pl.ds(row0, _ROWS)], osem1)
        o2 = pltpu.async_copy(zs3, oz_hbm.at[b, pl.ds(row0, _ROWS)], osem2)
        o3 = pltpu.async_copy(ds3, od_hbm.at[b, pl.ds(row0, _ROWS)], osem3)
        o1.wait()
        o2.wait()
        o3.wait()

    return k(verts)


def kernel(points, R, T):
    verts_view = jnp.einsum('bpi,bij->bpj', points, R) + T[:, None, :]
    verts = verts_view.reshape(-1)
    return _sc_rasterize(verts)


# speedup vs baseline: 1.0291x; 1.0291x over previous
import functools

import jax
import jax.numpy as jnp
from jax import lax
from jax.experimental import pallas as pl
from jax.experimental.pallas import tpu as pltpu
from jax.experimental.pallas import tpu_sc as plsc

_S = 64
_RAD = 0.05
_K = 8
_P = 2000
_ROWS = 4
_PIX = _ROWS * _S
_GRP = _P // 16
_INV32 = 0.03125
_EPS = 1e-3


def _sc_rasterize(verts):
    mesh = plsc.VectorSubcoreMesh(
        core_axis_name="c", subcore_axis_name="s", num_cores=2,
        num_subcores=16)

    @functools.partial(
        pl.kernel,
        mesh=mesh,
        compiler_params=pltpu.CompilerParams(needs_layout_passes=False),
        out_type=(
            jax.ShapeDtypeStruct((2, _S, _S, _K), jnp.int32),
            jax.ShapeDtypeStruct((2, _S, _S, _K), jnp.float32),
            jax.ShapeDtypeStruct((2, _S, _S, _K), jnp.float32),
        ),
        scratch_types=[
            pltpu.VMEM((3 * _P,), jnp.float32),
            pltpu.SemaphoreType.DMA,
            pltpu.SemaphoreType.DMA,
            pltpu.SemaphoreType.DMA,
            pltpu.SemaphoreType.DMA,
            pltpu.VMEM((16,), jnp.float32),
            pltpu.VMEM((16,), jnp.float32),
            pltpu.VMEM((16,), jnp.float32),
            pltpu.VMEM((16,), jnp.int32),
            pltpu.VMEM((_PIX * _K,), jnp.float32),
            pltpu.VMEM((_PIX * _K,), jnp.int32),
            pltpu.VMEM((_PIX * _K,), jnp.float32),
            pltpu.VMEM((_ROWS, _S, _K), jnp.float32),
            pltpu.VMEM((_ROWS, _S, _K), jnp.int32),
            pltpu.VMEM((_ROWS, _S, _K), jnp.float32),
        ],
    )
    def k(verts_hbm, oi_hbm, oz_hbm, od_hbm,
          pv, dsem, osem1, osem2, osem3, cx, cy, cz, ci, z8, i8, d8,
          zs3, is3, ds3):
        c = lax.axis_index("c")
        s = lax.axis_index("s")
        b = s & 1
        band = (s >> 1) + c * 8
        row0 = band * _ROWS

        pbase = b * (3 * _P)
        in_dma = pltpu.async_copy(verts_hbm.at[pl.ds(pbase, 3 * _P)], pv, dsem)

        lane = lax.iota(jnp.int32, 16)
        lm4 = lane & 3
        ld4 = lane >> 2
        rows_f = lax.convert_element_type(row0 + ld4, jnp.float32)
        pyv = 1.0 - (rows_f + 0.5) * _INV32
        row0_f = rows_f - lax.convert_element_type(ld4, jnp.float32)
        ymin = 1.0 - (row0_f + 3.5) * _INV32 - (_RAD + _EPS)
        ymax = 1.0 - (row0_f + 0.5) * _INV32 + (_RAD + _EPS)
        xlo = jnp.full((16,), -0.984375 - (_RAD + _EPS), jnp.float32)
        xhi = jnp.full((16,), 0.984375 + (_RAD + _EPS), jnp.float32)
        inf_v = jnp.full((16,), jnp.inf, jnp.float32)

        def init_body(i, carry):
            z8[pl.ds(i * 16, 16)] = inf_v
            return carry
        lax.fori_loop(0, _PIX * _K // 16, init_body, 0)
        in_dma.wait()

        r2v = jnp.full((16,), _RAD * _RAD, jnp.float32)

        def cand_body(i, carry):
            ib = lax.broadcast_in_dim(i, (16,), ())
            xs = plsc.load_gather(cx.at[...], [ib])
            ys = plsc.load_gather(cy.at[...], [ib])
            zs = plsc.load_gather(cz.at[...], [ib])
            iv = plsc.load_gather(ci.at[...], [ib])
            u = (1.0 - xs - _RAD) * 32.0 - 0.5 - _EPS
            ti = lax.convert_element_type(u, jnp.int32)
            ti = ti - jnp.where(lax.convert_element_type(ti, jnp.float32) > u,
                                jnp.int32(1), jnp.int32(0))
            cols = ti + 1 + lm4
            colf = lax.convert_element_type(cols, jnp.float32)
            pxv = 1.0 - (colf + 0.5) * _INV32
            dx = pxv - xs
            dy = pyv - ys
            d2 = dx * dx + dy * dy
            valid = (d2 < r2v) & (cols >= 0) & (cols < _S)
            nv = plsc.all_reduce_population_count(valid)[0]

            @pl.when(nv > 0)
            def _insert():
                addr = jnp.where(valid, (ld4 * _S + cols) * _K, jnp.int32(0))
                candz = zs
                candi = iv
                candd = d2
                for j in range(_K):
                    a = addr + j
                    curz = plsc.load_gather(z8.at[...], [a])
                    curi = plsc.load_gather(i8.at[...], [a])
                    curd = plsc.load_gather(d8.at[...], [a])
                    swap = candz < curz
                    newz = jnp.where(swap, candz, curz)
                    candz = jnp.where(swap, curz, candz)
                    newi = jnp.where(swap, candi, curi)
                    candi = jnp.where(swap, curi, candi)
                    newd = jnp.where(swap, candd, curd)
                    candd = jnp.where(swap, curd, candd)
                    plsc.store_scatter(z8.at[...], [a], newz, mask=valid)
                    plsc.store_scatter(i8.at[...], [a], newi, mask=valid)
                    plsc.store_scatter(d8.at[...], [a], newd, mask=valid)
                candz, candi, candd = None, None, None
            return carry

        def g_body(g, carry):
            off = g * 16
            vx = pv[pl.ds(off, 16)]
            vy = pv[pl.ds(_P + off, 16)]
            vz = pv[pl.ds(2 * _P + off, 16)]
            zsafe = jnp.where(jnp.abs(vz) < 1e-4, jnp.float32(1e-4), vz)
            m = ((vz > 0.0)
                 & (vy > ymin * zsafe) & (vy < ymax * zsafe)
                 & (vx > xlo * zsafe) & (vx < xhi * zsafe))
            n = plsc.all_reduce_population_count(m)[0]

            @pl.when(n > 0)
            def _emit():
                xn = vx / zsafe
                yn = vy / zsafe
                plsc.store_compressed(cx.at[...], xn, mask=m)
                plsc.store_compressed(cy.at[...], yn, mask=m)
                plsc.store_compressed(cz.at[...], vz, mask=m)
                plsc.store_compressed(ci.at[...], off + lane, mask=m)
                lax.fori_loop(0, n, cand_body, 0)
            return carry

        lax.fori_loop(0, _GRP, g_body, 0)

        kk_s = lane & (_K - 1)

        def fin_body(i, carry):
            sl = pl.ds(i * 16, 16)
            zv = z8[sl]
            empty = zv >= 3e38
            flat = i * 16 + lane
            pix = flat >> 3
            idxs = [pix >> 6, pix & (_S - 1), kk_s]
            plsc.store_scatter(zs3.at[...], idxs,
                               jnp.where(empty, jnp.float32(-1.0), zv))
            plsc.store_scatter(is3.at[...], idxs,
                               jnp.where(empty, jnp.int32(-1), i8[sl]))
            plsc.store_scatter(ds3.at[...], idxs,
                               jnp.where(empty, jnp.float32(-1.0), d8[sl]))
            return carry
        lax.fori_loop(0, _PIX * _K // 16, fin_body, 0)

        o1 = pltpu.async_copy(is3, oi_hbm.at[b, pl.ds(row0, _ROWS)], osem1)
        o2 = pltpu.async_copy(zs3, oz_hbm.at[b, pl.ds(row0, _ROWS)], osem2)
        o3 = pltpu.async_copy(ds3, od_hbm.at[b, pl.ds(row0, _ROWS)], osem3)
        o1.wait()
        o2.wait()
        o3.wait()

    return k(verts)


def kernel(points, R, T):
    verts_view = jnp.einsum('bpi,bij->bpj', points, R) + T[:, None, :]
    verts = jnp.transpose(verts_view, (0, 2, 1)).reshape(-1)
    return _sc_rasterize(verts)
